# Initial kernel scaffold; baseline (speedup 1.0000x reference)
import jax
import jax.numpy as jnp
from jax.experimental import pallas as pl


def kernel(output, activation, token_position):
    raise NotImplementedError("write your pallas kernel here")


# speedup vs baseline: 1.0501x; 1.0501x over previous
import functools

import jax
import jax.numpy as jnp
from jax import lax
from jax.experimental import pallas as pl
from jax.experimental.pallas import tpu as pltpu
from jax.experimental.pallas import tpu_sc as plsc

_B, _S, _D = 4, 8192, 2048
_NPOS = 64
_NC, _NS = 2, 16
_NW = _NC * _NS
_ROWS = _B * _NPOS
_RPW = _ROWS // _NW


@functools.cache
def _sc_scatter():
    @functools.partial(
        pl.kernel,
        mesh=plsc.VectorSubcoreMesh(
            core_axis_name="c", subcore_axis_name="s",
            num_cores=_NC, num_subcores=_NS,
        ),
        scratch_types=[
            pltpu.VMEM((_RPW,), jnp.int32),
            pltpu.VMEM((_RPW, _D), jnp.float32),
            pltpu.SemaphoreType.DMA,
        ],
    )
    def body(act_hbm, idx_hbm, out_hbm, idx_v, act_v, sem):
        w = lax.axis_index("s") * _NC + lax.axis_index("c")
        g = (w * _RPW) % _NPOS
        pltpu.sync_copy(idx_hbm.at[w], idx_v)
        pltpu.sync_copy(act_hbm.at[pl.ds(g, _RPW)], act_v)
        pltpu.async_copy(act_v, out_hbm.at[idx_v], sem).wait()

    return body


def kernel(output, activation, token_position):
    flat = output.reshape(_B * _S, _D)
    row_idx = (
        token_position[None, :].astype(jnp.int32)
        + (jnp.arange(_B, dtype=jnp.int32) * _S)[:, None]
    ).reshape(_NW, _RPW)
    out_ref = jax.new_ref(flat)
    _sc_scatter()(activation, row_idx, out_ref)
    return jax.freeze(out_ref).reshape(_B, _S, _D)
